# Initial kernel scaffold; baseline (speedup 1.0000x reference)
import jax
import jax.numpy as jnp
from jax.experimental import pallas as pl


def kernel(x, params, edge_index):
    raise NotImplementedError("write your pallas kernel here")


# speedup vs baseline: 3.1835x; 3.1835x over previous
import functools

import jax
import jax.numpy as jnp
from jax import lax
from jax.experimental import pallas as pl
from jax.experimental.pallas import tpu as pltpu
from jax.experimental.pallas import tpu_sc as plsc

_N = 50000
_E = 800000
_D = 64
_FF = 256

_NC = 2
_NS = 16

_SPLIT0 = 25024
_SPLIT1 = _N - _SPLIT0
_RB = 1560
_REM0 = (_SPLIT0 - _NS * _RB) // 8
_REM1 = (_SPLIT1 - _NS * _RB) // 8

_SPR = 25088
_ZROWS = _SPR // _NS
_TRASH = _SPR - 1

_CHUNK = 128
_CPT = 391
_EPAD = _NS * _CPT * _CHUNK


def _sc_segment_sum_body(h_hbm, src_hbm, dst_hbm, zeros_hbm, agg_hbm,
                         src_v, dst_v, idx_v, rows_v, spmem, sem):
    cid = lax.axis_index("c")
    tid = lax.axis_index("s")

    pltpu.sync_copy(zeros_hbm, spmem.at[pl.ds(tid * _ZROWS, _ZROWS)])
    plsc.subcore_barrier()

    base = cid * _SPLIT0
    hi = jnp.where(cid == 0, _SPLIT0, _N)

    def chunk(i, carry):
        off = (tid * _CPT + i) * _CHUNK
        pltpu.sync_copy(src_hbm.at[pl.ds(off, _CHUNK)], src_v)
        pltpu.sync_copy(dst_hbm.at[pl.ds(off, _CHUNK)], dst_v)
        gather = pltpu.async_copy(h_hbm.at[src_v], rows_v, sem)
        for j in range(_CHUNK // 16):
            d = dst_v[pl.ds(j * 16, 16)]
            ok = (d >= base) & (d < hi)
            idx_v[pl.ds(j * 16, 16)] = jnp.where(ok, d - base, _TRASH)
        gather.wait()
        pltpu.sync_copy(rows_v, spmem.at[idx_v], add=True)
        return carry

    lax.fori_loop(0, _CPT, chunk, 0)
    plsc.subcore_barrier()

    lo = tid * _RB
    pltpu.sync_copy(spmem.at[pl.ds(lo, _RB)], agg_hbm.at[pl.ds(base + lo, _RB)])
    rem = jnp.where(cid == 0, _REM0, _REM1)

    @pl.when(tid < rem)
    def _():
        r = _NS * _RB + tid * 8
        pltpu.sync_copy(spmem.at[pl.ds(r, 8)], agg_hbm.at[pl.ds(base + r, 8)])


_sc_segment_sum = functools.partial(
    pl.kernel,
    mesh=plsc.VectorSubcoreMesh(core_axis_name="c", subcore_axis_name="s",
                                num_cores=_NC, num_subcores=_NS),
    out_type=jax.ShapeDtypeStruct((_N, _D), jnp.float32),
    scratch_types=[
        pltpu.VMEM((_CHUNK,), jnp.int32),
        pltpu.VMEM((_CHUNK,), jnp.int32),
        pltpu.VMEM((_CHUNK,), jnp.int32),
        pltpu.VMEM((_CHUNK, _D), jnp.float32),
        pltpu.VMEM_SHARED((_SPR, _D), jnp.float32),
        pltpu.SemaphoreType.DMA,
    ],
    compiler_params=pltpu.CompilerParams(use_tc_tiling_on_sc=False),
)(_sc_segment_sum_body)


_BR = 2000


def _enc_body(x_ref, w_ref, b_ref, o_ref):
    o_ref[...] = jnp.maximum(x_ref[...] * w_ref[...] + b_ref[...], 0.0)


def _layer_norm(x, g, b):
    m = jnp.mean(x, axis=-1, keepdims=True)
    v = jnp.mean((x - m) ** 2, axis=-1, keepdims=True)
    return (x - m) / jnp.sqrt(v + 1e-5) * g + b


def _block_compute(h_ref, a_ref, W1, b1, W2, b2, g1, be1,
                   Wf1, bf1, Wf2, bf2, g2, be2):
    h = h_ref[...]
    z = h + a_ref[...]
    z = jnp.maximum(
        jnp.dot(z, W1[...], preferred_element_type=jnp.float32) + b1[...], 0.0)
    z = jnp.dot(z, W2[...], preferred_element_type=jnp.float32) + b2[...]
    h1 = _layer_norm(z + h, g1[...], be1[...])
    f = jnp.maximum(
        jnp.dot(h1, Wf1[...], preferred_element_type=jnp.float32) + bf1[...], 0.0)
    f = jnp.dot(f, Wf2[...], preferred_element_type=jnp.float32) + bf2[...]
    return _layer_norm(f + h1, g2[...], be2[...])


def _block_body(h_ref, a_ref, W1, b1, W2, b2, g1, be1,
                Wf1, bf1, Wf2, bf2, g2, be2, o_ref):
    o_ref[...] = _block_compute(h_ref, a_ref, W1, b1, W2, b2, g1, be1,
                                Wf1, bf1, Wf2, bf2, g2, be2)


def _block_last_body(h_ref, a_ref, W1, b1, W2, b2, g1, be1,
                     Wf1, bf1, Wf2, bf2, g2, be2, o_ref, s_ref, m_ref):
    h2 = _block_compute(h_ref, a_ref, W1, b1, W2, b2, g1, be1,
                        Wf1, bf1, Wf2, bf2, g2, be2)
    o_ref[...] = h2
    i = pl.program_id(0)

    @pl.when(i == 0)
    def _():
        s_ref[...] = jnp.zeros_like(s_ref)
        m_ref[...] = jnp.full_like(m_ref, -jnp.inf)

    s_ref[...] += jnp.sum(h2, axis=0, keepdims=True)
    m_ref[...] = jnp.maximum(m_ref[...], jnp.max(h2, axis=0, keepdims=True))

    @pl.when(i == pl.num_programs(0) - 1)
    def _():
        s_ref[...] = s_ref[...] * (1.0 / _N)


def _row_spec(shape):
    return pl.BlockSpec(shape, lambda i: (i, 0))


def _full_spec(shape):
    return pl.BlockSpec(shape, lambda i: (0, 0))


_GRID = _N // _BR

_enc_call = pl.pallas_call(
    _enc_body,
    grid=(_GRID,),
    in_specs=[_row_spec((_BR, 1)), _full_spec((1, _D)), _full_spec((1, _D))],
    out_specs=_row_spec((_BR, _D)),
    out_shape=jax.ShapeDtypeStruct((_N, _D), jnp.float32),
)

_W_SPECS = [
    _full_spec((_D, _D)), _full_spec((1, _D)),
    _full_spec((_D, _D)), _full_spec((1, _D)),
    _full_spec((1, _D)), _full_spec((1, _D)),
    _full_spec((_D, _FF)), _full_spec((1, _FF)),
    _full_spec((_FF, _D)), _full_spec((1, _D)),
    _full_spec((1, _D)), _full_spec((1, _D)),
]

_block_call = pl.pallas_call(
    _block_body,
    grid=(_GRID,),
    in_specs=[_row_spec((_BR, _D)), _row_spec((_BR, _D))] + _W_SPECS,
    out_specs=_row_spec((_BR, _D)),
    out_shape=jax.ShapeDtypeStruct((_N, _D), jnp.float32),
)

_block_last_call = pl.pallas_call(
    _block_last_body,
    grid=(_GRID,),
    in_specs=[_row_spec((_BR, _D)), _row_spec((_BR, _D))] + _W_SPECS,
    out_specs=[_row_spec((_BR, _D)), _full_spec((1, _D)), _full_spec((1, _D))],
    out_shape=[
        jax.ShapeDtypeStruct((_N, _D), jnp.float32),
        jax.ShapeDtypeStruct((1, _D), jnp.float32),
        jax.ShapeDtypeStruct((1, _D), jnp.float32),
    ],
)


def _block_weights(p):
    r = lambda a: a.reshape(1, -1)
    return (p["W1"], r(p["b1"]), p["W2"], r(p["b2"]), r(p["g1"]), r(p["be1"]),
            p["Wf1"], r(p["bf1"]), p["Wf2"], r(p["bf2"]), r(p["g2"]), r(p["be2"]))


def kernel(x, params, edge_index):
    src = edge_index[0]
    dst = edge_index[1]
    pad = _EPAD - _E
    src_p = jnp.concatenate([src, jnp.zeros((pad,), jnp.int32)])
    dst_p = jnp.concatenate([dst, jnp.full((pad,), _N, jnp.int32)])
    zeros = jnp.zeros((_ZROWS, _D), jnp.float32)

    h = _enc_call(x, params["enc_W"].reshape(1, _D),
                  params["enc_b"].reshape(1, _D))
    blocks = params["blocks"]
    for p in blocks[:-1]:
        agg = _sc_segment_sum(h, src_p, dst_p, zeros)
        h = _block_call(h, agg, *_block_weights(p))
    agg = _sc_segment_sum(h, src_p, dst_p, zeros)
    h, mean, mx = _block_last_call(h, agg, *_block_weights(blocks[-1]))
    graph_embedding = jnp.concatenate([mean, mx], axis=1)
    return (graph_embedding, h)


# speedup vs baseline: 8.4276x; 2.6473x over previous
import functools

import jax
import jax.numpy as jnp
from jax import lax
from jax.experimental import pallas as pl
from jax.experimental.pallas import tpu as pltpu
from jax.experimental.pallas import tpu_sc as plsc

_N = 50000
_E = 800000
_D = 64
_FF = 256

_NC = 2
_NS = 16

_HD = _D // 2

_SPR = 50016
_ZROWS = _SPR // _NS
_RB = 3120
_REMG = (_N - _NS * _RB) // 8

_CHUNK = 128
_K = 3
_NG = 131
_CPT = _K * _NG
_EPAD = _NS * _CPT * _CHUNK
_TCH = _NS * _CPT


def _sc_segment_sum_body(h_hbm, src2_hbm, dst_hbm, zeros_hbm, agg_hbm,
                         sidx, didx, rows, spmem, lsem, gsem, ssem):
    cid = lax.axis_index("c")
    tid = lax.axis_index("s")

    pltpu.sync_copy(zeros_hbm, spmem.at[pl.ds(tid * _ZROWS, _ZROWS)])

    cbase = tid * _CPT

    def load_idx(g, s):
        pltpu.async_copy(src2_hbm.at[cid, pl.ds(cbase + g * _K, _K)],
                         sidx[s], lsem[s])
        pltpu.async_copy(dst_hbm.at[pl.ds(cbase + g * _K, _K)],
                         didx[s], lsem[s])

    plsc.subcore_barrier()
    load_idx(0, 0)

    def superstep(i, carry):
        for u in range(4):
            g = i * 4 + u
            b = u & 1
            o = 1 - b
            s = u

            @pl.when(jnp.logical_and(g >= 2, g <= _NG + 1))
            def _():
                s2 = (u + 2) % 4
                for j in range(_K):
                    pltpu.make_async_copy(rows[b].at[j],
                                          spmem.at[didx[s2].at[j]],
                                          ssem[b]).wait()

            @pl.when(g < _NG)
            def _():
                pltpu.make_async_copy(
                    src2_hbm.at[cid, pl.ds(cbase + g * _K, _K)],
                    sidx[s], lsem[s]).wait()
                pltpu.make_async_copy(
                    dst_hbm.at[pl.ds(cbase + g * _K, _K)],
                    didx[s], lsem[s]).wait()
                for j in range(_K):
                    pltpu.async_copy(h_hbm.at[sidx[s].at[j]],
                                     rows[b].at[j], gsem[b])

            @pl.when(jnp.logical_and(g >= 1, g <= _NG))
            def _():
                s1 = (u + 3) % 4
                for j in range(_K):
                    pltpu.make_async_copy(h_hbm.at[sidx[s1].at[j]],
                                          rows[o].at[j], gsem[o]).wait()
                for j in range(_K):
                    pltpu.async_copy(rows[o].at[j],
                                     spmem.at[didx[s1].at[j]],
                                     ssem[o], add=True)

            @pl.when(g <= _NG - 2)
            def _():
                load_idx(g + 1, (u + 1) % 4)
        return carry

    lax.fori_loop(0, (_NG + 3) // 4 + 1, superstep, 0)
    plsc.subcore_barrier()

    lo = tid * _RB
    pltpu.sync_copy(spmem.at[pl.ds(lo, _RB)],
                    agg_hbm.at[cid, pl.ds(lo, _RB)])

    @pl.when(tid < _REMG)
    def _():
        r = _NS * _RB + tid * 8
        pltpu.sync_copy(spmem.at[pl.ds(r, 8)], agg_hbm.at[cid, pl.ds(r, 8)])


_sc_segment_sum = functools.partial(
    pl.kernel,
    mesh=plsc.VectorSubcoreMesh(core_axis_name="c", subcore_axis_name="s",
                                num_cores=_NC, num_subcores=_NS),
    out_type=jax.ShapeDtypeStruct((_NC, _N, _HD), jnp.float32),
    scratch_types=[
        [pltpu.VMEM((_K, _CHUNK), jnp.int32) for _ in range(4)],
        [pltpu.VMEM((_K, _CHUNK), jnp.int32) for _ in range(4)],
        [pltpu.VMEM((_K, _CHUNK, _HD), jnp.float32) for _ in range(2)],
        pltpu.VMEM_SHARED((_SPR, _HD), jnp.float32),
        [pltpu.SemaphoreType.DMA for _ in range(4)],
        [pltpu.SemaphoreType.DMA for _ in range(2)],
        [pltpu.SemaphoreType.DMA for _ in range(2)],
    ],
    compiler_params=pltpu.CompilerParams(use_tc_tiling_on_sc=False),
)(_sc_segment_sum_body)


_BR = 2000


def _enc_body(x_ref, w_ref, b_ref, o_ref):
    o_ref[...] = jnp.maximum(x_ref[...] * w_ref[...] + b_ref[...], 0.0)


def _layer_norm(x, g, b):
    m = jnp.mean(x, axis=-1, keepdims=True)
    v = jnp.mean((x - m) ** 2, axis=-1, keepdims=True)
    return (x - m) / jnp.sqrt(v + 1e-5) * g + b


def _block_compute(h_ref, a_ref, W1, b1, W2, b2, g1, be1,
                   Wf1, bf1, Wf2, bf2, g2, be2):
    h = h_ref[...]
    agg = jnp.concatenate([a_ref[0], a_ref[1]], axis=-1)
    z = h + agg
    z = jnp.maximum(
        jnp.dot(z, W1[...], preferred_element_type=jnp.float32) + b1[...], 0.0)
    z = jnp.dot(z, W2[...], preferred_element_type=jnp.float32) + b2[...]
    h1 = _layer_norm(z + h, g1[...], be1[...])
    f = jnp.maximum(
        jnp.dot(h1, Wf1[...], preferred_element_type=jnp.float32) + bf1[...], 0.0)
    f = jnp.dot(f, Wf2[...], preferred_element_type=jnp.float32) + bf2[...]
    return _layer_norm(f + h1, g2[...], be2[...])


def _block_body(h_ref, a_ref, W1, b1, W2, b2, g1, be1,
                Wf1, bf1, Wf2, bf2, g2, be2, o_ref):
    o_ref[...] = _block_compute(h_ref, a_ref, W1, b1, W2, b2, g1, be1,
                                Wf1, bf1, Wf2, bf2, g2, be2)


def _block_last_body(h_ref, a_ref, W1, b1, W2, b2, g1, be1,
                     Wf1, bf1, Wf2, bf2, g2, be2, o_ref, s_ref, m_ref):
    h2 = _block_compute(h_ref, a_ref, W1, b1, W2, b2, g1, be1,
                        Wf1, bf1, Wf2, bf2, g2, be2)
    o_ref[...] = h2
    i = pl.program_id(0)

    @pl.when(i == 0)
    def _():
        s_ref[...] = jnp.zeros_like(s_ref)
        m_ref[...] = jnp.full_like(m_ref, -jnp.inf)

    s_ref[...] += jnp.sum(h2, axis=0, keepdims=True)
    m_ref[...] = jnp.maximum(m_ref[...], jnp.max(h2, axis=0, keepdims=True))

    @pl.when(i == pl.num_programs(0) - 1)
    def _():
        s_ref[...] = s_ref[...] * (1.0 / _N)


def _row_spec(shape):
    return pl.BlockSpec(shape, lambda i: (i, 0))


def _full_spec(shape):
    return pl.BlockSpec(shape, lambda i: (0, 0))


_GRID = _N // _BR

_enc_call = pl.pallas_call(
    _enc_body,
    grid=(_GRID,),
    in_specs=[_row_spec((_BR, 1)), _full_spec((1, _D)), _full_spec((1, _D))],
    out_specs=_row_spec((_BR, _D)),
    out_shape=jax.ShapeDtypeStruct((_N, _D), jnp.float32),
)

_W_SPECS = [
    _full_spec((_D, _D)), _full_spec((1, _D)),
    _full_spec((_D, _D)), _full_spec((1, _D)),
    _full_spec((1, _D)), _full_spec((1, _D)),
    _full_spec((_D, _FF)), _full_spec((1, _FF)),
    _full_spec((_FF, _D)), _full_spec((1, _D)),
    _full_spec((1, _D)), _full_spec((1, _D)),
]

_agg_spec = pl.BlockSpec((_NC, _BR, _HD), lambda i: (0, i, 0))

_block_call = pl.pallas_call(
    _block_body,
    grid=(_GRID,),
    in_specs=[_row_spec((_BR, _D)), _agg_spec] + _W_SPECS,
    out_specs=_row_spec((_BR, _D)),
    out_shape=jax.ShapeDtypeStruct((_N, _D), jnp.float32),
)

_block_last_call = pl.pallas_call(
    _block_last_body,
    grid=(_GRID,),
    in_specs=[_row_spec((_BR, _D)), _agg_spec] + _W_SPECS,
    out_specs=[_row_spec((_BR, _D)), _full_spec((1, _D)), _full_spec((1, _D))],
    out_shape=[
        jax.ShapeDtypeStruct((_N, _D), jnp.float32),
        jax.ShapeDtypeStruct((1, _D), jnp.float32),
        jax.ShapeDtypeStruct((1, _D), jnp.float32),
    ],
)


def _block_weights(p):
    r = lambda a: a.reshape(1, -1)
    return (p["W1"], r(p["b1"]), p["W2"], r(p["b2"]), r(p["g1"]), r(p["be1"]),
            p["Wf1"], r(p["bf1"]), p["Wf2"], r(p["bf2"]), r(p["g2"]), r(p["be2"]))


def kernel(x, params, edge_index):
    src = edge_index[0]
    dst = edge_index[1]
    pad = _EPAD - _E
    src_p = jnp.concatenate([src, jnp.zeros((pad,), jnp.int32)])
    dst_p = jnp.concatenate([dst, jnp.full((pad,), _N, jnp.int32)])
    src2 = jnp.stack([src_p * 2, src_p * 2 + 1]).reshape(_NC, _TCH, _CHUNK)
    dst_p = dst_p.reshape(_TCH, _CHUNK)
    zeros = jnp.zeros((_ZROWS, _HD), jnp.float32)

    h = _enc_call(x, params["enc_W"].reshape(1, _D),
                  params["enc_b"].reshape(1, _D))
    blocks = params["blocks"]
    for p in blocks[:-1]:
        agg = _sc_segment_sum(h.reshape(_NC * _N, _HD), src2, dst_p, zeros)
        h = _block_call(h, agg, *_block_weights(p))
    agg = _sc_segment_sum(h.reshape(_NC * _N, _HD), src2, dst_p, zeros)
    h, mean, mx = _block_last_call(h, agg, *_block_weights(blocks[-1]))
    graph_embedding = jnp.concatenate([mean, mx], axis=1)
    return (graph_embedding, h)
